# Initial kernel scaffold; baseline (speedup 1.0000x reference)
import jax
import jax.numpy as jnp
from jax.experimental import pallas as pl


def kernel(x, edge_index, num_nodes, params):
    raise NotImplementedError("write your pallas kernel here")


# speedup vs baseline: 1.0302x; 1.0302x over previous
import functools

import jax
import jax.numpy as jnp
import numpy as np
from jax import lax
from jax.experimental import pallas as pl
from jax.experimental.pallas import tpu as pltpu
from jax.experimental.pallas import tpu_sc as plsc

N = 10000
NP = 10240
E0 = 160000
H = 32
NW = 32
CAPW = 114688
E_MAX = NW * CAPW
HALF = CAPW // 2
DUMP = 10200
DUMPVAL = (DUMP << 14) | DUMP
E0P = 163840
ROWS_W = NP // NW
NC16 = NP // 16

@functools.lru_cache(maxsize=1)
def _mesh():
    return plsc.VectorSubcoreMesh(core_axis_name="c", subcore_axis_name="s")


def _wid():
    return lax.axis_index("s") * 2 + lax.axis_index("c")


def _iota16():
    return lax.broadcasted_iota(jnp.int32, (16,), 0)


def _ln_mlp(p, x):
    x = jnp.maximum(jnp.dot(x, p["W1"], preferred_element_type=jnp.float32) + p["b1"], 0.0)
    x = jnp.maximum(jnp.dot(x, p["W2"], preferred_element_type=jnp.float32) + p["b2"], 0.0)
    x = jnp.maximum(jnp.dot(x, p["W3"], preferred_element_type=jnp.float32) + p["b3"], 0.0)
    x = jnp.dot(x, p["W4"], preferred_element_type=jnp.float32) + p["b4"]
    mu = jnp.mean(x, axis=-1, keepdims=True)
    var = jnp.mean((x - mu) ** 2, axis=-1, keepdims=True)
    return (x - mu) * jax.lax.rsqrt(var + 1e-5) * p["g"] + p["be"]


def _mlp_args(params, name):
    p = params[name]
    return [p["W1"], p["b1"].reshape(1, -1), p["W2"], p["b2"].reshape(1, -1),
            p["W3"], p["b3"].reshape(1, -1), p["W4"], p["b4"].reshape(1, -1),
            p["g"].reshape(1, -1), p["be"].reshape(1, -1)]


def _take_mlp(refs, i0):
    names = ["W1", "b1", "W2", "b2", "W3", "b3", "W4", "b4", "g", "be"]
    return {n: refs[i0 + k][...] for k, n in enumerate(names)}


def _wspecs(n):
    return [pl.BlockSpec(None, lambda *_: (0, 0)) for _ in range(n)]


def _enc_body(x_ref, *refs):
    out_ref = refs[-1]
    p = _take_mlp(refs, 0)
    y = _ln_mlp(p, x_ref[...][:, :9])
    out_ref[...] = jnp.pad(y, ((0, 0), (0, 128 - H)))


def _encoder(x, params):
    xp = jnp.pad(x, ((0, 0), (0, 7)))
    blk = 2000
    return pl.pallas_call(
        _enc_body,
        grid=(E0 // blk,),
        in_specs=[pl.BlockSpec((blk, 16), lambda i: (i, 0))] + _wspecs(10),
        out_specs=pl.BlockSpec((blk, 128), lambda i: (i, 0)),
        out_shape=jax.ShapeDtypeStruct((E0, 128), jnp.float32),
    )(xp, *_mlp_args(params, "eb"))


def _mask_body(a_ref, b_ref, aij_ref, pk_ref, c16_ref, acc_ref):
    i, j, k = pl.program_id(0), pl.program_id(1), pl.program_id(2)

    @pl.when(k == 0)
    def _():
        acc_ref[...] = jnp.zeros_like(acc_ref)

    acc_ref[...] += jnp.dot(a_ref[...], b_ref[...],
                            preferred_element_type=jnp.float32)

    @pl.when(k == pl.num_programs(2) - 1)
    def _():
        blk = acc_ref.shape[0]
        rows = i * blk + lax.broadcasted_iota(jnp.int32, acc_ref.shape, 0)
        cols = j * blk + lax.broadcasted_iota(jnp.int32, acc_ref.shape, 1)
        two = (acc_ref[...] > 0.0).astype(jnp.int32) * (rows != cols).astype(jnp.int32)
        mi = jnp.minimum((aij_ref[...] > 0).astype(jnp.int32) + two, 1)
        lanemod = lax.broadcasted_iota(jnp.int32, acc_ref.shape, 1) % 16
        x = mi
        for sh in (1, 2, 4, 8):
            r = jnp.pad(x, ((0, 0), (sh, 0)))[:, :-sh]
            x = x + r * (lanemod >= sh).astype(jnp.int32)
        p_excl = x - mi
        c16_ref[...] = x.reshape(blk, blk // 16, 16)[:, :, 15][None]
        d = lanemod - p_excl
        packed = jnp.zeros_like(mi)
        for dd in range(16):
            src = cols * mi * (d == dd).astype(jnp.int32)
            if dd:
                src = jnp.pad(src, ((0, 0), (0, dd)))[:, dd:]
            packed = packed + src
        pk_ref[...] = packed


def _two_hop_mask(a_bf):
    blk = 512
    g = NP // blk
    return pl.pallas_call(
        _mask_body,
        grid=(g, g, g),
        in_specs=[
            pl.BlockSpec((blk, blk), lambda i, j, k: (i, k)),
            pl.BlockSpec((blk, blk), lambda i, j, k: (k, j)),
            pl.BlockSpec((blk, blk), lambda i, j, k: (i, j)),
        ],
        out_specs=[pl.BlockSpec((blk, blk), lambda i, j, k: (i, j)),
                   pl.BlockSpec((1, blk, blk // 16), lambda i, j, k: (j, i, 0))],
        out_shape=[jax.ShapeDtypeStruct((NP, NP), jnp.int32),
                   jax.ShapeDtypeStruct((NP // blk, NP, blk // 16), jnp.int32)],
        scratch_shapes=[pltpu.VMEM((blk, blk), jnp.float32)],
        compiler_params=pltpu.CompilerParams(
            dimension_semantics=("parallel", "arbitrary", "arbitrary")),
    )(a_bf, a_bf, a_bf)


def _compact_kernel(pk_hbm, c16_hbm, edges_hbm, cnts_hbm,
                    stage_v, row_v, cnt_v, t16_v):
    w = _wid()
    dump = jnp.full((16,), DUMPVAL, jnp.int32)
    nstage = HALF // 16

    def _init_stage():
        def bi(q, _):
            stage_v[pl.ds(q * 16, 16)] = dump
            return 0
        lax.fori_loop(0, nstage + 1, bi, 0)

    _init_stage()
    pltpu.sync_copy(stage_v.at[pl.ds(0, HALF)], edges_hbm.at[pl.ds(w * CAPW, HALF)])
    pltpu.sync_copy(stage_v.at[pl.ds(0, HALF)],
                    edges_hbm.at[pl.ds(w * CAPW + HALF, HALF)])

    def row_body(r, carry):
        gr = w * ROWS_W + r
        pltpu.sync_copy(pk_hbm.at[gr], row_v)
        pltpu.sync_copy(c16_hbm.at[gr], cnt_v)
        rowterm = gr << 14

        def c40_body(c40, carry2):
            cur2, nf2 = carry2
            cntv = cnt_v[pl.ds(c40 * 16, 16)]
            for u in range(16):
                cnt = cntv[u]
                off = (c40 * 16 + u) * 16
                v = row_v[pl.ds(off, 16)] + rowterm
                stage_v[pl.ds(cur2, 16)] = v
                cur3 = cur2 + cnt
                crossed = cur3 >= HALF

                def _flush():
                    carry_v = stage_v[pl.ds(HALF, 16)]

                    @pl.when(nf2 == 0)
                    def _():
                        pltpu.sync_copy(stage_v.at[pl.ds(0, HALF)],
                                        edges_hbm.at[pl.ds(w * CAPW, HALF)])

                    @pl.when(nf2 == 1)
                    def _():
                        pltpu.sync_copy(stage_v.at[pl.ds(0, HALF)],
                                        edges_hbm.at[pl.ds(w * CAPW + HALF, HALF)])

                    _init_stage()
                    stage_v[pl.ds(0, 16)] = carry_v

                pl.when(crossed)(_flush)
                cur2 = jnp.where(crossed, cur3 - HALF, cur3)
                nf2 = nf2 + crossed.astype(jnp.int32)
            return (cur2, nf2)

        return lax.fori_loop(0, NC16 // 16, c40_body, carry)

    cur, nf = lax.fori_loop(0, ROWS_W, row_body, (0, 0))
    stage_v[pl.ds(cur, 16)] = dump

    @pl.when(nf == 0)
    def _():
        pltpu.sync_copy(stage_v.at[pl.ds(0, HALF)],
                        edges_hbm.at[pl.ds(w * CAPW, HALF)])

    @pl.when(nf == 1)
    def _():
        pltpu.sync_copy(stage_v.at[pl.ds(0, HALF)],
                        edges_hbm.at[pl.ds(w * CAPW + HALF, HALF)])

    total = jnp.minimum(jnp.minimum(nf, 2) * HALF + cur, CAPW)
    t16_v[pl.ds(0, 16)] = jnp.zeros((16,), jnp.int32) + total
    pltpu.sync_copy(t16_v, cnts_hbm.at[w])


def _compact(pk, c16):
    k = functools.partial(
        pl.kernel,
        out_type=(jax.ShapeDtypeStruct((E_MAX + 8192,), jnp.int32),
                  jax.ShapeDtypeStruct((NW, 16), jnp.int32)),
        mesh=_mesh(),
        scratch_types=[pltpu.VMEM((HALF + 16,), jnp.int32),
                       pltpu.VMEM((NP,), jnp.int32),
                       pltpu.VMEM((NC16,), jnp.int32),
                       pltpu.VMEM((16,), jnp.int32)],
    )(_compact_kernel)
    return k(pk, c16)


def _extract_kernel(edges_hbm, g_hbm, e0_hbm, e1_hbm,
                    g_v, win0_v, win1_v, b0_v, b1_v):
    u = _wid()
    o0 = u * 5120
    pltpu.sync_copy(g_hbm, g_v)
    gs = []
    for q in range(3):
        gq = g_v[pl.ds(q * 16, 16)]
        for t in range(16):
            if q * 16 + t <= NW:
                gs.append(gq[t])
    w0 = jnp.int32(0)
    for jj in range(1, NW + 1):
        w0 = w0 + (gs[jj] <= o0).astype(jnp.int32)
    w0 = jnp.clip(w0, 0, NW - 1)
    w1 = jnp.minimum(w0 + 1, NW - 1)
    gw0 = jnp.int32(0)
    tstar = jnp.int32(0)
    for jj in range(NW + 1):
        gw0 = jnp.where(w0 == jj, gs[jj], gw0)
        tstar = jnp.where(w0 + 1 == jj, gs[jj], tstar)
    kstar = jnp.clip(tstar - o0, 0, 1 << 29)

    src0 = jnp.clip(w0 * CAPW + (o0 - gw0), 0, E_MAX + 8192 - 5136)
    src0a = pl.multiple_of((src0 >> 3) << 3, 8)
    shift0 = src0 - src0a
    pltpu.sync_copy(edges_hbm.at[pl.ds(src0a, 5136)], win0_v)
    src1 = pl.multiple_of(w1 * CAPW, 8)
    pltpu.sync_copy(edges_hbm.at[pl.ds(src1, 5136)], win1_v)

    def body(kk, _):
        off0 = shift0 + kk * 16
        off1 = jnp.clip(kk * 16 - kstar, 0, 5120)
        v0 = win0_v[pl.ds(off0, 16)]
        v1 = win1_v[pl.ds(off1, 16)]
        sel = (_iota16() + kk * 16) >= kstar
        v = jnp.where(sel, v1, v0)
        b0_v[pl.ds(kk * 16, 16)] = v >> 14
        b1_v[pl.ds(kk * 16, 16)] = v & 16383
        return 0

    lax.fori_loop(0, 320, body, 0)
    dst = pl.multiple_of(o0, 8)
    pltpu.sync_copy(b0_v, e0_hbm.at[pl.ds(dst, 5120)])
    pltpu.sync_copy(b1_v, e1_hbm.at[pl.ds(dst, 5120)])


def _extract(edges, g48):
    k = functools.partial(
        pl.kernel,
        out_type=(jax.ShapeDtypeStruct((E0P,), jnp.int32),
                  jax.ShapeDtypeStruct((E0P,), jnp.int32)),
        mesh=_mesh(),
        scratch_types=[pltpu.VMEM((48,), jnp.int32),
                       pltpu.VMEM((5136,), jnp.int32),
                       pltpu.VMEM((5136,), jnp.int32),
                       pltpu.VMEM((5120,), jnp.int32),
                       pltpu.VMEM((5120,), jnp.int32)],
    )(_extract_kernel)
    return k(edges, g48)


def _zero_spmem(acc_sh, z_v, rows_per_copy):
    sid = lax.axis_index("s")

    @pl.when(sid == 0)
    def _():
        zz = jnp.zeros((16,), jnp.float32)
        for r in range(rows_per_copy):
            for q in range(128 // 16):
                z_v[r, pl.ds(q * 16, 16)] = zz

        def cp(b, _):
            pltpu.sync_copy(z_v, acc_sh.at[pl.ds(b * rows_per_copy, rows_per_copy)])
            return 0
        lax.fori_loop(0, NP // rows_per_copy, cp, 0)

    plsc.subcore_barrier()


def _enc_scatter_kernel(xe_hbm, e0_hbm, e1_hbm, out_hbm,
                        idx_v, val_v, z_v, acc_sh):
    cid = lax.axis_index("c")
    w = _wid()
    _zero_spmem(acc_sh, z_v, 64)
    base = w * (E0 // NW)

    def batch(b, _):
        pltpu.sync_copy(xe_hbm.at[pl.ds(base + b * 40, 40)], val_v)
        pltpu.sync_copy(e0_hbm.at[pl.ds(base + b * 40, 40)], idx_v)
        pltpu.sync_copy(val_v, acc_sh.at[idx_v], add=True)
        pltpu.sync_copy(e1_hbm.at[pl.ds(base + b * 40, 40)], idx_v)
        pltpu.sync_copy(val_v, acc_sh.at[idx_v], add=True)
        return 0

    lax.fori_loop(0, E0 // NW // 40, batch, 0)
    plsc.subcore_barrier()
    sid = lax.axis_index("s")

    @pl.when(sid == 0)
    def _():
        pltpu.sync_copy(acc_sh, out_hbm.at[cid])


def _enc_scatter(xe, e0, e1):
    k = functools.partial(
        pl.kernel,
        out_type=jax.ShapeDtypeStruct((2, NP, 128), jnp.float32),
        mesh=_mesh(),
        scratch_types=[pltpu.VMEM((40,), jnp.int32),
                       pltpu.VMEM((40, 128), jnp.float32),
                       pltpu.VMEM((64, 128), jnp.float32),
                       pltpu.VMEM_SHARED((NP, 128), jnp.float32)],
    )(_enc_scatter_kernel)
    return k(xe, e0, e1)


def _pair_gather_kernel(tab_hbm, edges_hbm, fa_hbm, fb_hbm,
                        pk_v, ia_v, ib_v, ga_v, gb_v, sem):
    w = _wid()
    B = 128

    def batch(b, _):
        base = w * CAPW + b * B
        pltpu.sync_copy(edges_hbm.at[pl.ds(base, B)], pk_v)

        def unpk(q, _2):
            v = pk_v[pl.ds(q * 16, 16)]
            ia_v[pl.ds(q * 16, 16)] = v >> 14
            ib_v[pl.ds(q * 16, 16)] = v & 16383
            return 0
        lax.fori_loop(0, B // 16, unpk, 0)
        pltpu.async_copy(tab_hbm.at[ia_v], ga_v, sem).wait()
        pltpu.sync_copy(ga_v, fa_hbm.at[pl.ds(base, B)])
        pltpu.async_copy(tab_hbm.at[ib_v], gb_v, sem).wait()
        pltpu.sync_copy(gb_v, fb_hbm.at[pl.ds(base, B)])
        return 0

    lax.fori_loop(0, CAPW // B, batch, 0)


def _pair_gather(tab, edges):
    k = functools.partial(
        pl.kernel,
        out_type=(jax.ShapeDtypeStruct((E_MAX, 128), jnp.float32),
                  jax.ShapeDtypeStruct((E_MAX, 128), jnp.float32)),
        mesh=_mesh(),
        scratch_types=[pltpu.VMEM((128,), jnp.int32),
                       pltpu.VMEM((128,), jnp.int32),
                       pltpu.VMEM((128,), jnp.int32),
                       pltpu.VMEM((128, 128), jnp.float32),
                       pltpu.VMEM((128, 128), jnp.float32),
                       pltpu.SemaphoreType.DMA],
    )(_pair_gather_kernel)
    return k(tab, edges)


def _pair_scatter_kernel(xew_hbm, edges_hbm, out_hbm,
                         pk_v, ia_v, ib_v, val_v, z_v, acc_sh):
    cid = lax.axis_index("c")
    w = _wid()
    _zero_spmem(acc_sh, z_v, 64)
    B = 256

    def batch(b, _):
        base = w * CAPW + b * B
        pltpu.sync_copy(xew_hbm.at[pl.ds(base, B)], val_v)
        pltpu.sync_copy(edges_hbm.at[pl.ds(base, B)], pk_v)

        def sub(s, _2):
            def unpk(q, _3):
                v = pk_v[pl.ds(s * 64 + q * 16, 16)]
                ia_v[pl.ds(q * 16, 16)] = v >> 14
                ib_v[pl.ds(q * 16, 16)] = v & 16383
                return 0
            lax.fori_loop(0, 4, unpk, 0)
            pltpu.sync_copy(val_v.at[pl.ds(s * 64, 64)],
                            acc_sh.at[ia_v], add=True)
            pltpu.sync_copy(val_v.at[pl.ds(s * 64, 64)],
                            acc_sh.at[ib_v], add=True)
            return 0
        lax.fori_loop(0, B // 64, sub, 0)
        return 0

    lax.fori_loop(0, CAPW // B, batch, 0)
    plsc.subcore_barrier()
    sid = lax.axis_index("s")

    @pl.when(sid == 0)
    def _():
        pltpu.sync_copy(acc_sh, out_hbm.at[cid])


def _pair_scatter(xew, edges):
    k = functools.partial(
        pl.kernel,
        out_type=jax.ShapeDtypeStruct((2, NP, 128), jnp.float32),
        mesh=_mesh(),
        scratch_types=[pltpu.VMEM((256,), jnp.int32),
                       pltpu.VMEM((64,), jnp.int32),
                       pltpu.VMEM((64,), jnp.int32),
                       pltpu.VMEM((256, 128), jnp.float32),
                       pltpu.VMEM((64, 128), jnp.float32),
                       pltpu.VMEM_SHARED((NP, 128), jnp.float32)],
    )(_pair_scatter_kernel)
    return k(xew, edges)


def _dec_gather_kernel(tab_hbm, e0_hbm, e1_hbm, fa_hbm, fb_hbm,
                       ia_v, ga_v, sem):
    u = _wid()
    B = 128

    def batch(b, _):
        base = u * 5120 + b * B
        pltpu.sync_copy(e0_hbm.at[pl.ds(base, B)], ia_v)
        pltpu.async_copy(tab_hbm.at[ia_v], ga_v, sem).wait()
        pltpu.sync_copy(ga_v, fa_hbm.at[pl.ds(base, B)])
        pltpu.sync_copy(e1_hbm.at[pl.ds(base, B)], ia_v)
        pltpu.async_copy(tab_hbm.at[ia_v], ga_v, sem).wait()
        pltpu.sync_copy(ga_v, fb_hbm.at[pl.ds(base, B)])
        return 0

    lax.fori_loop(0, 5120 // B, batch, 0)


def _dec_gather(tab, e0, e1):
    k = functools.partial(
        pl.kernel,
        out_type=(jax.ShapeDtypeStruct((E0P, 128), jnp.float32),
                  jax.ShapeDtypeStruct((E0P, 128), jnp.float32)),
        mesh=_mesh(),
        scratch_types=[pltpu.VMEM((128,), jnp.int32),
                       pltpu.VMEM((128, 128), jnp.float32),
                       pltpu.SemaphoreType.DMA],
    )(_dec_gather_kernel)
    return k(tab, e0, e1)


def _node_body(acc_ref, xn_ref, *refs):
    out_ref = refs[-1]
    p = _take_mlp(refs, 0)
    agg = acc_ref[0] + acc_ref[1]
    xn = xn_ref[...][:, :H]
    y = _ln_mlp(p, jnp.concatenate([xn, agg[:, :H]], axis=1))
    out_ref[...] = jnp.pad(xn + y, ((0, 0), (0, 128 - H)))


def _node_mlp(acc, xn_tab, params, name):
    blk = 1024
    return pl.pallas_call(
        _node_body,
        grid=(NP // blk,),
        in_specs=[pl.BlockSpec((2, blk, 128), lambda i: (0, i, 0)),
                  pl.BlockSpec((blk, 128), lambda i: (i, 0))] + _wspecs(10),
        out_specs=pl.BlockSpec((blk, 128), lambda i: (i, 0)),
        out_shape=jax.ShapeDtypeStruct((NP, 128), jnp.float32),
    )(acc, xn_tab, *_mlp_args(params, name))


def _node0_body(acc_ref, *refs):
    out_ref = refs[-1]
    p = _take_mlp(refs, 0)
    pre = acc_ref[0][:, :H] + acc_ref[1][:, :H]
    y = _ln_mlp(p, pre)
    out_ref[...] = jnp.pad(y, ((0, 0), (0, 128 - H)))


def _node0_mlp(acc, params):
    blk = 1024
    return pl.pallas_call(
        _node0_body,
        grid=(NP // blk,),
        in_specs=[pl.BlockSpec((2, blk, 128), lambda i: (0, i, 0))] + _wspecs(10),
        out_specs=pl.BlockSpec((blk, 128), lambda i: (i, 0)),
        out_shape=jax.ShapeDtypeStruct((NP, 128), jnp.float32),
    )(acc, *_mlp_args(params, "nb"))


def _pair0_body(fa_ref, fb_ref, *refs):
    xew_ref, xe1_ref = refs[-2], refs[-1]
    p1 = _take_mlp(refs, 0)
    p2 = _take_mlp(refs, 10)
    a = fa_ref[...][:, :H]
    b = fb_ref[...][:, :H]
    ab = jnp.concatenate([a, b], axis=1)
    xe0 = _ln_mlp(p1, ab)
    xe_new = _ln_mlp(p2, jnp.concatenate([ab, xe0], axis=1))
    xew_ref[...] = jnp.pad(xe_new, ((0, 0), (0, 128 - H)))
    xe1_ref[...] = xe0 + xe_new


def _pair0(fa, fb, params):
    blk = 2048
    return pl.pallas_call(
        _pair0_body,
        grid=(E_MAX // blk,),
        in_specs=[pl.BlockSpec((blk, 128), lambda i: (i, 0)),
                  pl.BlockSpec((blk, 128), lambda i: (i, 0))] + _wspecs(20),
        out_specs=[pl.BlockSpec((blk, 128), lambda i: (i, 0)),
                   pl.BlockSpec((blk, H), lambda i: (i, 0))],
        out_shape=[jax.ShapeDtypeStruct((E_MAX, 128), jnp.float32),
                   jax.ShapeDtypeStruct((E_MAX, H), jnp.float32)],
    )(fa, fb, *_mlp_args(params, "eb1"), *_mlp_args(params, "gn_eb_0"))


def _pair1_body(fa_ref, fb_ref, xe1_ref, *refs):
    xew_ref = refs[-1]
    p = _take_mlp(refs, 0)
    a = fa_ref[...][:, :H]
    b = fb_ref[...][:, :H]
    xe_new = _ln_mlp(p, jnp.concatenate([a, b, xe1_ref[...]], axis=1))
    xew_ref[...] = jnp.pad(xe_new, ((0, 0), (0, 128 - H)))


def _pair1(fa, fb, xe1, params):
    blk = 2048
    return pl.pallas_call(
        _pair1_body,
        grid=(E_MAX // blk,),
        in_specs=[pl.BlockSpec((blk, 128), lambda i: (i, 0)),
                  pl.BlockSpec((blk, 128), lambda i: (i, 0)),
                  pl.BlockSpec((blk, H), lambda i: (i, 0))] + _wspecs(10),
        out_specs=pl.BlockSpec((blk, 128), lambda i: (i, 0)),
        out_shape=jax.ShapeDtypeStruct((E_MAX, 128), jnp.float32),
    )(fa, fb, xe1, *_mlp_args(params, "gn_eb_1"))


def _dec_body(fa_ref, fb_ref, *refs):
    out_ref = refs[-1]
    p = dict(_take_mlp(refs, 0))
    a = fa_ref[...][:, :H]
    b = fb_ref[...][:, :H]
    x = jnp.concatenate([a, b], axis=1)
    x = jnp.maximum(jnp.dot(x, p["W1"], preferred_element_type=jnp.float32) + p["b1"], 0.0)
    x = jnp.maximum(jnp.dot(x, p["W2"], preferred_element_type=jnp.float32) + p["b2"], 0.0)
    x = jnp.maximum(jnp.dot(x, p["W3"], preferred_element_type=jnp.float32) + p["b3"], 0.0)
    x = jnp.dot(x, p["W4"], preferred_element_type=jnp.float32) + p["b4"]
    y3 = x[:, :3]
    mu = jnp.mean(y3, axis=-1, keepdims=True)
    var = jnp.mean((y3 - mu) ** 2, axis=-1, keepdims=True)
    y = (y3 - mu) * jax.lax.rsqrt(var + 1e-5) * p["g"][:, :3] + p["be"][:, :3]
    out_ref[...] = jnp.pad(y, ((0, 0), (0, 5)))


def _decoder(fa, fb, params):
    p = params["dec"]
    args = [p["W1"], p["b1"].reshape(1, -1), p["W2"], p["b2"].reshape(1, -1),
            p["W3"], p["b3"].reshape(1, -1),
            jnp.pad(p["W4"], ((0, 0), (0, 5))), jnp.pad(p["b4"], (0, 5)).reshape(1, -1),
            jnp.pad(p["g"], (0, 5)).reshape(1, -1), jnp.pad(p["be"], (0, 5)).reshape(1, -1)]
    blk = 2048
    return pl.pallas_call(
        _dec_body,
        grid=(E0P // blk,),
        in_specs=[pl.BlockSpec((blk, 128), lambda i: (i, 0)),
                  pl.BlockSpec((blk, 128), lambda i: (i, 0))] + _wspecs(10),
        out_specs=pl.BlockSpec((blk, 8), lambda i: (i, 0)),
        out_shape=jax.ShapeDtypeStruct((E0P, 8), jnp.float32),
    )(fa, fb, *args)


def kernel(x, edge_index, num_nodes, params):
    src = edge_index[0].astype(jnp.int32)
    dst = edge_index[1].astype(jnp.int32)

    a_bf = jnp.zeros((NP, NP), jnp.bfloat16).at[src, dst].set(jnp.bfloat16(1))

    pk, c16 = _two_hop_mask(a_bf)
    c16f = jnp.moveaxis(c16, 0, 1).reshape(NP, NC16)
    edges, cnts16 = _compact(pk, c16f)
    cnts = cnts16[:, 0]
    g = jnp.concatenate([jnp.zeros((1,), jnp.int32), jnp.cumsum(cnts)])
    nnz = g[NW]
    g48 = jnp.concatenate([g.astype(jnp.int32),
                           jnp.full((48 - NW - 1,), 1 << 30, jnp.int32)])
    e0p, e1p = _extract(edges, g48)
    guard = jnp.arange(E0P, dtype=jnp.int32) < nnz
    e0p = jnp.where(guard, e0p, 0)
    e1p = jnp.where(guard, e1p, 0)

    xe = _encoder(x, params)
    xn_pre = _enc_scatter(xe, e0p[:E0], e1p[:E0])
    xn_tab = _node0_mlp(xn_pre, params)

    fa0, fb0 = _pair_gather(xn_tab, edges)
    xew0, xe1c = _pair0(fa0, fb0, params)
    acc0 = _pair_scatter(xew0, edges)
    xn1_tab = _node_mlp(acc0, xn_tab, params, "gn_nb_0")

    fa1, fb1 = _pair_gather(xn1_tab, edges)
    xew1 = _pair1(fa1, fb1, xe1c, params)
    acc1 = _pair_scatter(xew1, edges)
    xn2_tab = _node_mlp(acc1, xn1_tab, params, "gn_nb_1")

    dfa, dfb = _dec_gather(xn2_tab, e0p, e1p)
    outp = _decoder(dfa, dfb, params)
    return outp[:E0, :3]


# speedup vs baseline: 1.0364x; 1.0059x over previous
import functools

import jax
import jax.numpy as jnp
import numpy as np
from jax import lax
from jax.experimental import pallas as pl
from jax.experimental.pallas import tpu as pltpu
from jax.experimental.pallas import tpu_sc as plsc

N = 10000
NP = 10240
E0 = 160000
H = 32
NW = 32
CAPW = 114688
E_MAX = NW * CAPW
HALF = CAPW // 2
DUMP = 10200
DUMPVAL = (DUMP << 14) | DUMP
E0P = 163840
ROWS_W = NP // NW
NC16 = NP // 16

@functools.lru_cache(maxsize=1)
def _mesh():
    return plsc.VectorSubcoreMesh(core_axis_name="c", subcore_axis_name="s")


def _wid():
    return lax.axis_index("s") * 2 + lax.axis_index("c")


def _iota16():
    return lax.broadcasted_iota(jnp.int32, (16,), 0)


def _ln_mlp(p, x):
    x = jnp.maximum(jnp.dot(x, p["W1"], preferred_element_type=jnp.float32) + p["b1"], 0.0)
    x = jnp.maximum(jnp.dot(x, p["W2"], preferred_element_type=jnp.float32) + p["b2"], 0.0)
    x = jnp.maximum(jnp.dot(x, p["W3"], preferred_element_type=jnp.float32) + p["b3"], 0.0)
    x = jnp.dot(x, p["W4"], preferred_element_type=jnp.float32) + p["b4"]
    mu = jnp.mean(x, axis=-1, keepdims=True)
    var = jnp.mean((x - mu) ** 2, axis=-1, keepdims=True)
    return (x - mu) * jax.lax.rsqrt(var + 1e-5) * p["g"] + p["be"]


def _mlp_args(params, name):
    p = params[name]
    return [p["W1"], p["b1"].reshape(1, -1), p["W2"], p["b2"].reshape(1, -1),
            p["W3"], p["b3"].reshape(1, -1), p["W4"], p["b4"].reshape(1, -1),
            p["g"].reshape(1, -1), p["be"].reshape(1, -1)]


def _take_mlp(refs, i0):
    names = ["W1", "b1", "W2", "b2", "W3", "b3", "W4", "b4", "g", "be"]
    return {n: refs[i0 + k][...] for k, n in enumerate(names)}


def _wspecs(n):
    return [pl.BlockSpec(None, lambda *_: (0, 0)) for _ in range(n)]


def _enc_body(x_ref, *refs):
    out_ref = refs[-1]
    p = _take_mlp(refs, 0)
    y = _ln_mlp(p, x_ref[...][:, :9])
    out_ref[...] = jnp.pad(y, ((0, 0), (0, 128 - H)))


def _encoder(x, params):
    xp = jnp.pad(x, ((0, 0), (0, 7)))
    blk = 2000
    return pl.pallas_call(
        _enc_body,
        grid=(E0 // blk,),
        in_specs=[pl.BlockSpec((blk, 16), lambda i: (i, 0))] + _wspecs(10),
        out_specs=pl.BlockSpec((blk, 128), lambda i: (i, 0)),
        out_shape=jax.ShapeDtypeStruct((E0, 128), jnp.float32),
    )(xp, *_mlp_args(params, "eb"))


def _mask_body(a_ref, b_ref, aij_ref, pk_ref, c16_ref, acc_ref):
    i, j, k = pl.program_id(0), pl.program_id(1), pl.program_id(2)

    @pl.when(k == 0)
    def _():
        acc_ref[...] = jnp.zeros_like(acc_ref)

    acc_ref[...] += jnp.dot(a_ref[...], b_ref[...],
                            preferred_element_type=jnp.float32)

    @pl.when(k == pl.num_programs(2) - 1)
    def _():
        blk = acc_ref.shape[0]
        rows = i * blk + lax.broadcasted_iota(jnp.int32, acc_ref.shape, 0)
        cols = j * blk + lax.broadcasted_iota(jnp.int32, acc_ref.shape, 1)
        two = (acc_ref[...] > 0.0).astype(jnp.int32) * (rows != cols).astype(jnp.int32)
        mi = jnp.minimum((aij_ref[...] > 0).astype(jnp.int32) + two, 1)
        lanemod = lax.broadcasted_iota(jnp.int32, acc_ref.shape, 1) % 16
        x = mi
        for sh in (1, 2, 4, 8):
            r = jnp.pad(x, ((0, 0), (sh, 0)))[:, :-sh]
            x = x + r * (lanemod >= sh).astype(jnp.int32)
        p_excl = x - mi
        c16_ref[...] = x.reshape(blk, blk // 16, 16)[:, :, 15][None]
        d = lanemod - p_excl
        packed = jnp.zeros_like(mi)
        for dd in range(16):
            src = cols * mi * (d == dd).astype(jnp.int32)
            if dd:
                src = jnp.pad(src, ((0, 0), (0, dd)))[:, dd:]
            packed = packed + src
        pk_ref[...] = packed


def _two_hop_mask(a_bf):
    blk = 512
    g = NP // blk
    return pl.pallas_call(
        _mask_body,
        grid=(g, g, g),
        in_specs=[
            pl.BlockSpec((blk, blk), lambda i, j, k: (i, k)),
            pl.BlockSpec((blk, blk), lambda i, j, k: (k, j)),
            pl.BlockSpec((blk, blk), lambda i, j, k: (i, j)),
        ],
        out_specs=[pl.BlockSpec((blk, blk), lambda i, j, k: (i, j)),
                   pl.BlockSpec((1, blk, blk // 16), lambda i, j, k: (j, i, 0))],
        out_shape=[jax.ShapeDtypeStruct((NP, NP), jnp.int32),
                   jax.ShapeDtypeStruct((NP // blk, NP, blk // 16), jnp.int32)],
        scratch_shapes=[pltpu.VMEM((blk, blk), jnp.float32)],
        compiler_params=pltpu.CompilerParams(
            dimension_semantics=("parallel", "arbitrary", "arbitrary")),
    )(a_bf, a_bf, a_bf)


def _compact_kernel(pk_hbm, c16_hbm, edges_hbm, cnts_hbm,
                    stage_v, row_v, cnt_v, t16_v):
    w = _wid()
    dump = jnp.full((16,), DUMPVAL, jnp.int32)
    nstage = HALF // 16

    def _init_stage():
        def bi(q, _):
            stage_v[pl.ds(q * 16, 16)] = dump
            return 0
        lax.fori_loop(0, nstage + 1, bi, 0)

    _init_stage()
    pltpu.sync_copy(stage_v.at[pl.ds(0, HALF)], edges_hbm.at[pl.ds(w * CAPW, HALF)])
    pltpu.sync_copy(stage_v.at[pl.ds(0, HALF)],
                    edges_hbm.at[pl.ds(w * CAPW + HALF, HALF)])

    def row_body(r, carry):
        gr = w * ROWS_W + r
        pltpu.sync_copy(pk_hbm.at[gr], row_v)
        pltpu.sync_copy(c16_hbm.at[gr], cnt_v)
        rowterm = gr << 14

        def c40_body(c40, carry2):
            cur2, nf2 = carry2
            cntv = cnt_v[pl.ds(c40 * 16, 16)]
            for u in range(16):
                cnt = cntv[u]
                off = (c40 * 16 + u) * 16
                v = row_v[pl.ds(off, 16)] + rowterm
                stage_v[pl.ds(cur2, 16)] = v
                cur3 = cur2 + cnt
                crossed = cur3 >= HALF

                def _flush():
                    carry_v = stage_v[pl.ds(HALF, 16)]

                    @pl.when(nf2 == 0)
                    def _():
                        pltpu.sync_copy(stage_v.at[pl.ds(0, HALF)],
                                        edges_hbm.at[pl.ds(w * CAPW, HALF)])

                    @pl.when(nf2 == 1)
                    def _():
                        pltpu.sync_copy(stage_v.at[pl.ds(0, HALF)],
                                        edges_hbm.at[pl.ds(w * CAPW + HALF, HALF)])

                    _init_stage()
                    stage_v[pl.ds(0, 16)] = carry_v

                pl.when(crossed)(_flush)
                cur2 = jnp.where(crossed, cur3 - HALF, cur3)
                nf2 = nf2 + crossed.astype(jnp.int32)
            return (cur2, nf2)

        return lax.fori_loop(0, NC16 // 16, c40_body, carry)

    cur, nf = lax.fori_loop(0, ROWS_W, row_body, (0, 0))
    stage_v[pl.ds(cur, 16)] = dump

    @pl.when(nf == 0)
    def _():
        pltpu.sync_copy(stage_v.at[pl.ds(0, HALF)],
                        edges_hbm.at[pl.ds(w * CAPW, HALF)])

    @pl.when(nf == 1)
    def _():
        pltpu.sync_copy(stage_v.at[pl.ds(0, HALF)],
                        edges_hbm.at[pl.ds(w * CAPW + HALF, HALF)])

    total = jnp.minimum(jnp.minimum(nf, 2) * HALF + cur, CAPW)
    t16_v[pl.ds(0, 16)] = jnp.zeros((16,), jnp.int32) + total
    pltpu.sync_copy(t16_v, cnts_hbm.at[w])


def _compact(pk, c16):
    k = functools.partial(
        pl.kernel,
        out_type=(jax.ShapeDtypeStruct((E_MAX + 8192,), jnp.int32),
                  jax.ShapeDtypeStruct((NW, 16), jnp.int32)),
        mesh=_mesh(),
        scratch_types=[pltpu.VMEM((HALF + 16,), jnp.int32),
                       pltpu.VMEM((NP,), jnp.int32),
                       pltpu.VMEM((NC16,), jnp.int32),
                       pltpu.VMEM((16,), jnp.int32)],
    )(_compact_kernel)
    return k(pk, c16)


def _extract_kernel(edges_hbm, g_hbm, e0_hbm, e1_hbm,
                    g_v, win0_v, win1_v, b0_v, b1_v):
    u = _wid()
    o0 = u * 5120
    pltpu.sync_copy(g_hbm, g_v)
    gs = []
    for q in range(3):
        gq = g_v[pl.ds(q * 16, 16)]
        for t in range(16):
            if q * 16 + t <= NW:
                gs.append(gq[t])
    w0 = jnp.int32(0)
    for jj in range(1, NW + 1):
        w0 = w0 + (gs[jj] <= o0).astype(jnp.int32)
    w0 = jnp.clip(w0, 0, NW - 1)
    w1 = jnp.minimum(w0 + 1, NW - 1)
    gw0 = jnp.int32(0)
    tstar = jnp.int32(0)
    for jj in range(NW + 1):
        gw0 = jnp.where(w0 == jj, gs[jj], gw0)
        tstar = jnp.where(w0 + 1 == jj, gs[jj], tstar)
    kstar = jnp.clip(tstar - o0, 0, 1 << 29)

    src0 = jnp.clip(w0 * CAPW + (o0 - gw0), 0, E_MAX + 8192 - 5136)
    src0a = pl.multiple_of((src0 >> 3) << 3, 8)
    shift0 = src0 - src0a
    pltpu.sync_copy(edges_hbm.at[pl.ds(src0a, 5136)], win0_v)
    src1 = pl.multiple_of(w1 * CAPW, 8)
    pltpu.sync_copy(edges_hbm.at[pl.ds(src1, 5136)], win1_v)

    def body(kk, _):
        off0 = shift0 + kk * 16
        off1 = jnp.clip(kk * 16 - kstar, 0, 5120)
        v0 = win0_v[pl.ds(off0, 16)]
        v1 = win1_v[pl.ds(off1, 16)]
        sel = (_iota16() + kk * 16) >= kstar
        v = jnp.where(sel, v1, v0)
        b0_v[pl.ds(kk * 16, 16)] = v >> 14
        b1_v[pl.ds(kk * 16, 16)] = v & 16383
        return 0

    lax.fori_loop(0, 320, body, 0)
    dst = pl.multiple_of(o0, 8)
    pltpu.sync_copy(b0_v, e0_hbm.at[pl.ds(dst, 5120)])
    pltpu.sync_copy(b1_v, e1_hbm.at[pl.ds(dst, 5120)])


def _extract(edges, g48):
    k = functools.partial(
        pl.kernel,
        out_type=(jax.ShapeDtypeStruct((E0P,), jnp.int32),
                  jax.ShapeDtypeStruct((E0P,), jnp.int32)),
        mesh=_mesh(),
        scratch_types=[pltpu.VMEM((48,), jnp.int32),
                       pltpu.VMEM((5136,), jnp.int32),
                       pltpu.VMEM((5136,), jnp.int32),
                       pltpu.VMEM((5120,), jnp.int32),
                       pltpu.VMEM((5120,), jnp.int32)],
    )(_extract_kernel)
    return k(edges, g48)


def _zero_spmem(acc_sh, z_v, rows_per_copy):
    sid = lax.axis_index("s")

    @pl.when(sid == 0)
    def _():
        zz = jnp.zeros((16,), jnp.float32)
        for r in range(rows_per_copy):
            for q in range(128 // 16):
                z_v[r, pl.ds(q * 16, 16)] = zz

        def cp(b, _):
            pltpu.sync_copy(z_v, acc_sh.at[pl.ds(b * rows_per_copy, rows_per_copy)])
            return 0
        lax.fori_loop(0, NP // rows_per_copy, cp, 0)

    plsc.subcore_barrier()


def _enc_scatter_kernel(xe_hbm, e0_hbm, e1_hbm, out_hbm,
                        idx_v, val_v, z_v, acc_sh):
    cid = lax.axis_index("c")
    w = _wid()
    _zero_spmem(acc_sh, z_v, 64)
    base = w * (E0 // NW)

    def batch(b, _):
        pltpu.sync_copy(xe_hbm.at[pl.ds(base + b * 40, 40)], val_v)
        pltpu.sync_copy(e0_hbm.at[pl.ds(base + b * 40, 40)], idx_v)
        pltpu.sync_copy(val_v, acc_sh.at[idx_v], add=True)
        pltpu.sync_copy(e1_hbm.at[pl.ds(base + b * 40, 40)], idx_v)
        pltpu.sync_copy(val_v, acc_sh.at[idx_v], add=True)
        return 0

    lax.fori_loop(0, E0 // NW // 40, batch, 0)
    plsc.subcore_barrier()
    sid = lax.axis_index("s")

    @pl.when(sid == 0)
    def _():
        pltpu.sync_copy(acc_sh, out_hbm.at[cid])


def _enc_scatter(xe, e0, e1):
    k = functools.partial(
        pl.kernel,
        out_type=jax.ShapeDtypeStruct((2, NP, 128), jnp.float32),
        mesh=_mesh(),
        scratch_types=[pltpu.VMEM((40,), jnp.int32),
                       pltpu.VMEM((40, 128), jnp.float32),
                       pltpu.VMEM((64, 128), jnp.float32),
                       pltpu.VMEM_SHARED((NP, 128), jnp.float32)],
    )(_enc_scatter_kernel)
    return k(xe, e0, e1)


def _pair_gather_kernel(tab_hbm, edges_hbm, fa_hbm, fb_hbm,
                        pk_v, ia_v, ib_v, ga_v, gb_v, sem):
    w = _wid()
    B = 128
    nbat = CAPW // B
    niter = (nbat + 2) // 3

    def batch3(bb, _):
        descs = []
        bases = []
        for k in range(3):
            b = jnp.minimum(bb * 3 + k, nbat - 1)
            base = pl.multiple_of(w * CAPW + b * B, 8)
            bases.append(base)
            pltpu.sync_copy(edges_hbm.at[pl.ds(base, B)], pk_v.at[k])

            def unpk(q, _2, k=k):
                v = pk_v[k, pl.ds(q * 16, 16)]
                ia_v[k, pl.ds(q * 16, 16)] = v >> 14
                ib_v[k, pl.ds(q * 16, 16)] = v & 16383
                return 0
            lax.fori_loop(0, B // 16, unpk, 0)
            descs.append(pltpu.async_copy(tab_hbm.at[ia_v.at[k]], ga_v.at[k], sem))
            descs.append(pltpu.async_copy(tab_hbm.at[ib_v.at[k]], gb_v.at[k], sem))
        for d in descs:
            d.wait()
        for k in range(3):
            pltpu.sync_copy(ga_v.at[k], fa_hbm.at[pl.ds(bases[k], B)])
            pltpu.sync_copy(gb_v.at[k], fb_hbm.at[pl.ds(bases[k], B)])
        return 0

    lax.fori_loop(0, niter, batch3, 0)


def _pair_gather(tab, edges):
    k = functools.partial(
        pl.kernel,
        out_type=(jax.ShapeDtypeStruct((E_MAX, 128), jnp.float32),
                  jax.ShapeDtypeStruct((E_MAX, 128), jnp.float32)),
        mesh=_mesh(),
        scratch_types=[pltpu.VMEM((3, 128), jnp.int32),
                       pltpu.VMEM((3, 128), jnp.int32),
                       pltpu.VMEM((3, 128), jnp.int32),
                       pltpu.VMEM((3, 128, 128), jnp.float32),
                       pltpu.VMEM((3, 128, 128), jnp.float32),
                       pltpu.SemaphoreType.DMA],
    )(_pair_gather_kernel)
    return k(tab, edges)


def _pair_scatter_kernel(xew_hbm, edges_hbm, out_hbm,
                         pk_v, ia_v, ib_v, val_v, z_v, acc_sh):
    cid = lax.axis_index("c")
    w = _wid()
    _zero_spmem(acc_sh, z_v, 64)
    B = 256

    def batch(b, _):
        base = w * CAPW + b * B
        pltpu.sync_copy(xew_hbm.at[pl.ds(base, B)], val_v)
        pltpu.sync_copy(edges_hbm.at[pl.ds(base, B)], pk_v)

        def sub(s, _2):
            def unpk(q, _3):
                v = pk_v[pl.ds(s * 64 + q * 16, 16)]
                ia_v[pl.ds(q * 16, 16)] = v >> 14
                ib_v[pl.ds(q * 16, 16)] = v & 16383
                return 0
            lax.fori_loop(0, 4, unpk, 0)
            pltpu.sync_copy(val_v.at[pl.ds(s * 64, 64)],
                            acc_sh.at[ia_v], add=True)
            pltpu.sync_copy(val_v.at[pl.ds(s * 64, 64)],
                            acc_sh.at[ib_v], add=True)
            return 0
        lax.fori_loop(0, B // 64, sub, 0)
        return 0

    lax.fori_loop(0, CAPW // B, batch, 0)
    plsc.subcore_barrier()
    sid = lax.axis_index("s")

    @pl.when(sid == 0)
    def _():
        pltpu.sync_copy(acc_sh, out_hbm.at[cid])


def _pair_scatter(xew, edges):
    k = functools.partial(
        pl.kernel,
        out_type=jax.ShapeDtypeStruct((2, NP, 128), jnp.float32),
        mesh=_mesh(),
        scratch_types=[pltpu.VMEM((256,), jnp.int32),
                       pltpu.VMEM((64,), jnp.int32),
                       pltpu.VMEM((64,), jnp.int32),
                       pltpu.VMEM((256, 128), jnp.float32),
                       pltpu.VMEM((64, 128), jnp.float32),
                       pltpu.VMEM_SHARED((NP, 128), jnp.float32)],
    )(_pair_scatter_kernel)
    return k(xew, edges)


def _dec_gather_kernel(tab_hbm, e0_hbm, e1_hbm, fa_hbm, fb_hbm,
                       ia_v, ib_v, ga_v, gb_v, sem):
    u = _wid()
    B = 128
    nbat = 5120 // B
    niter = (nbat + 2) // 3

    def batch3(bb, _):
        descs = []
        bases = []
        for k in range(3):
            b = jnp.minimum(bb * 3 + k, nbat - 1)
            base = pl.multiple_of(u * 5120 + b * B, 8)
            bases.append(base)
            pltpu.sync_copy(e0_hbm.at[pl.ds(base, B)], ia_v.at[k])
            pltpu.sync_copy(e1_hbm.at[pl.ds(base, B)], ib_v.at[k])
            descs.append(pltpu.async_copy(tab_hbm.at[ia_v.at[k]], ga_v.at[k], sem))
            descs.append(pltpu.async_copy(tab_hbm.at[ib_v.at[k]], gb_v.at[k], sem))
        for d in descs:
            d.wait()
        for k in range(3):
            pltpu.sync_copy(ga_v.at[k], fa_hbm.at[pl.ds(bases[k], B)])
            pltpu.sync_copy(gb_v.at[k], fb_hbm.at[pl.ds(bases[k], B)])
        return 0

    lax.fori_loop(0, niter, batch3, 0)


def _dec_gather(tab, e0, e1):
    k = functools.partial(
        pl.kernel,
        out_type=(jax.ShapeDtypeStruct((E0P, 128), jnp.float32),
                  jax.ShapeDtypeStruct((E0P, 128), jnp.float32)),
        mesh=_mesh(),
        scratch_types=[pltpu.VMEM((3, 128), jnp.int32),
                       pltpu.VMEM((3, 128), jnp.int32),
                       pltpu.VMEM((3, 128, 128), jnp.float32),
                       pltpu.VMEM((3, 128, 128), jnp.float32),
                       pltpu.SemaphoreType.DMA],
    )(_dec_gather_kernel)
    return k(tab, e0, e1)


def _node_body(acc_ref, xn_ref, *refs):
    out_ref = refs[-1]
    p = _take_mlp(refs, 0)
    agg = acc_ref[0] + acc_ref[1]
    xn = xn_ref[...][:, :H]
    y = _ln_mlp(p, jnp.concatenate([xn, agg[:, :H]], axis=1))
    out_ref[...] = jnp.pad(xn + y, ((0, 0), (0, 128 - H)))


def _node_mlp(acc, xn_tab, params, name):
    blk = 1024
    return pl.pallas_call(
        _node_body,
        grid=(NP // blk,),
        in_specs=[pl.BlockSpec((2, blk, 128), lambda i: (0, i, 0)),
                  pl.BlockSpec((blk, 128), lambda i: (i, 0))] + _wspecs(10),
        out_specs=pl.BlockSpec((blk, 128), lambda i: (i, 0)),
        out_shape=jax.ShapeDtypeStruct((NP, 128), jnp.float32),
    )(acc, xn_tab, *_mlp_args(params, name))


def _node0_body(acc_ref, *refs):
    out_ref = refs[-1]
    p = _take_mlp(refs, 0)
    pre = acc_ref[0][:, :H] + acc_ref[1][:, :H]
    y = _ln_mlp(p, pre)
    out_ref[...] = jnp.pad(y, ((0, 0), (0, 128 - H)))


def _node0_mlp(acc, params):
    blk = 1024
    return pl.pallas_call(
        _node0_body,
        grid=(NP // blk,),
        in_specs=[pl.BlockSpec((2, blk, 128), lambda i: (0, i, 0))] + _wspecs(10),
        out_specs=pl.BlockSpec((blk, 128), lambda i: (i, 0)),
        out_shape=jax.ShapeDtypeStruct((NP, 128), jnp.float32),
    )(acc, *_mlp_args(params, "nb"))


def _pair0_body(fa_ref, fb_ref, *refs):
    xew_ref, xe1_ref = refs[-2], refs[-1]
    p1 = _take_mlp(refs, 0)
    p2 = _take_mlp(refs, 10)
    a = fa_ref[...][:, :H]
    b = fb_ref[...][:, :H]
    ab = jnp.concatenate([a, b], axis=1)
    xe0 = _ln_mlp(p1, ab)
    xe_new = _ln_mlp(p2, jnp.concatenate([ab, xe0], axis=1))
    xew_ref[...] = jnp.pad(xe_new, ((0, 0), (0, 128 - H)))
    xe1_ref[...] = xe0 + xe_new


def _pair0(fa, fb, params):
    blk = 2048
    return pl.pallas_call(
        _pair0_body,
        grid=(E_MAX // blk,),
        in_specs=[pl.BlockSpec((blk, 128), lambda i: (i, 0)),
                  pl.BlockSpec((blk, 128), lambda i: (i, 0))] + _wspecs(20),
        out_specs=[pl.BlockSpec((blk, 128), lambda i: (i, 0)),
                   pl.BlockSpec((blk, H), lambda i: (i, 0))],
        out_shape=[jax.ShapeDtypeStruct((E_MAX, 128), jnp.float32),
                   jax.ShapeDtypeStruct((E_MAX, H), jnp.float32)],
    )(fa, fb, *_mlp_args(params, "eb1"), *_mlp_args(params, "gn_eb_0"))


def _pair1_body(fa_ref, fb_ref, xe1_ref, *refs):
    xew_ref = refs[-1]
    p = _take_mlp(refs, 0)
    a = fa_ref[...][:, :H]
    b = fb_ref[...][:, :H]
    xe_new = _ln_mlp(p, jnp.concatenate([a, b, xe1_ref[...]], axis=1))
    xew_ref[...] = jnp.pad(xe_new, ((0, 0), (0, 128 - H)))


def _pair1(fa, fb, xe1, params):
    blk = 2048
    return pl.pallas_call(
        _pair1_body,
        grid=(E_MAX // blk,),
        in_specs=[pl.BlockSpec((blk, 128), lambda i: (i, 0)),
                  pl.BlockSpec((blk, 128), lambda i: (i, 0)),
                  pl.BlockSpec((blk, H), lambda i: (i, 0))] + _wspecs(10),
        out_specs=pl.BlockSpec((blk, 128), lambda i: (i, 0)),
        out_shape=jax.ShapeDtypeStruct((E_MAX, 128), jnp.float32),
    )(fa, fb, xe1, *_mlp_args(params, "gn_eb_1"))


def _dec_body(fa_ref, fb_ref, *refs):
    out_ref = refs[-1]
    p = dict(_take_mlp(refs, 0))
    a = fa_ref[...][:, :H]
    b = fb_ref[...][:, :H]
    x = jnp.concatenate([a, b], axis=1)
    x = jnp.maximum(jnp.dot(x, p["W1"], preferred_element_type=jnp.float32) + p["b1"], 0.0)
    x = jnp.maximum(jnp.dot(x, p["W2"], preferred_element_type=jnp.float32) + p["b2"], 0.0)
    x = jnp.maximum(jnp.dot(x, p["W3"], preferred_element_type=jnp.float32) + p["b3"], 0.0)
    x = jnp.dot(x, p["W4"], preferred_element_type=jnp.float32) + p["b4"]
    y3 = x[:, :3]
    mu = jnp.mean(y3, axis=-1, keepdims=True)
    var = jnp.mean((y3 - mu) ** 2, axis=-1, keepdims=True)
    y = (y3 - mu) * jax.lax.rsqrt(var + 1e-5) * p["g"][:, :3] + p["be"][:, :3]
    out_ref[...] = jnp.pad(y, ((0, 0), (0, 5)))


def _decoder(fa, fb, params):
    p = params["dec"]
    args = [p["W1"], p["b1"].reshape(1, -1), p["W2"], p["b2"].reshape(1, -1),
            p["W3"], p["b3"].reshape(1, -1),
            jnp.pad(p["W4"], ((0, 0), (0, 5))), jnp.pad(p["b4"], (0, 5)).reshape(1, -1),
            jnp.pad(p["g"], (0, 5)).reshape(1, -1), jnp.pad(p["be"], (0, 5)).reshape(1, -1)]
    blk = 2048
    return pl.pallas_call(
        _dec_body,
        grid=(E0P // blk,),
        in_specs=[pl.BlockSpec((blk, 128), lambda i: (i, 0)),
                  pl.BlockSpec((blk, 128), lambda i: (i, 0))] + _wspecs(10),
        out_specs=pl.BlockSpec((blk, 8), lambda i: (i, 0)),
        out_shape=jax.ShapeDtypeStruct((E0P, 8), jnp.float32),
    )(fa, fb, *args)


def kernel(x, edge_index, num_nodes, params):
    src = edge_index[0].astype(jnp.int32)
    dst = edge_index[1].astype(jnp.int32)

    a_bf = jnp.zeros((NP, NP), jnp.bfloat16).at[src, dst].set(jnp.bfloat16(1))

    pk, c16 = _two_hop_mask(a_bf)
    c16f = jnp.moveaxis(c16, 0, 1).reshape(NP, NC16)
    edges, cnts16 = _compact(pk, c16f)
    cnts = cnts16[:, 0]
    g = jnp.concatenate([jnp.zeros((1,), jnp.int32), jnp.cumsum(cnts)])
    nnz = g[NW]
    g48 = jnp.concatenate([g.astype(jnp.int32),
                           jnp.full((48 - NW - 1,), 1 << 30, jnp.int32)])
    e0p, e1p = _extract(edges, g48)
    guard = jnp.arange(E0P, dtype=jnp.int32) < nnz
    e0p = jnp.where(guard, e0p, 0)
    e1p = jnp.where(guard, e1p, 0)

    xe = _encoder(x, params)
    xn_pre = _enc_scatter(xe, e0p[:E0], e1p[:E0])
    xn_tab = _node0_mlp(xn_pre, params)

    fa0, fb0 = _pair_gather(xn_tab, edges)
    xew0, xe1c = _pair0(fa0, fb0, params)
    acc0 = _pair_scatter(xew0, edges)
    xn1_tab = _node_mlp(acc0, xn_tab, params, "gn_nb_0")

    fa1, fb1 = _pair_gather(xn1_tab, edges)
    xew1 = _pair1(fa1, fb1, xe1c, params)
    acc1 = _pair_scatter(xew1, edges)
    xn2_tab = _node_mlp(acc1, xn1_tab, params, "gn_nb_1")

    dfa, dfb = _dec_gather(xn2_tab, e0p, e1p)
    outp = _decoder(dfa, dfb, params)
    return outp[:E0, :3]


# speedup vs baseline: 1.0366x; 1.0002x over previous
import functools

import jax
import jax.numpy as jnp
import numpy as np
from jax import lax
from jax.experimental import pallas as pl
from jax.experimental.pallas import tpu as pltpu
from jax.experimental.pallas import tpu_sc as plsc

N = 10000
NP = 10240
E0 = 160000
H = 32
NW = 32
CAPW = 114688
E_MAX = NW * CAPW
HALF = CAPW // 2
DUMP = 10200
DUMPVAL = (DUMP << 14) | DUMP
E0P = 163840
ROWS_W = NP // NW
NC16 = NP // 16

@functools.lru_cache(maxsize=1)
def _mesh():
    return plsc.VectorSubcoreMesh(core_axis_name="c", subcore_axis_name="s")


def _wid():
    return lax.axis_index("s") * 2 + lax.axis_index("c")


def _iota16():
    return lax.broadcasted_iota(jnp.int32, (16,), 0)


def _ln_mlp(p, x):
    x = jnp.maximum(jnp.dot(x, p["W1"], preferred_element_type=jnp.float32) + p["b1"], 0.0)
    x = jnp.maximum(jnp.dot(x, p["W2"], preferred_element_type=jnp.float32) + p["b2"], 0.0)
    x = jnp.maximum(jnp.dot(x, p["W3"], preferred_element_type=jnp.float32) + p["b3"], 0.0)
    x = jnp.dot(x, p["W4"], preferred_element_type=jnp.float32) + p["b4"]
    mu = jnp.mean(x, axis=-1, keepdims=True)
    var = jnp.mean((x - mu) ** 2, axis=-1, keepdims=True)
    return (x - mu) * jax.lax.rsqrt(var + 1e-5) * p["g"] + p["be"]


def _mlp_args(params, name):
    p = params[name]
    return [p["W1"], p["b1"].reshape(1, -1), p["W2"], p["b2"].reshape(1, -1),
            p["W3"], p["b3"].reshape(1, -1), p["W4"], p["b4"].reshape(1, -1),
            p["g"].reshape(1, -1), p["be"].reshape(1, -1)]


def _take_mlp(refs, i0):
    names = ["W1", "b1", "W2", "b2", "W3", "b3", "W4", "b4", "g", "be"]
    return {n: refs[i0 + k][...] for k, n in enumerate(names)}


def _wspecs(n):
    return [pl.BlockSpec(None, lambda *_: (0, 0)) for _ in range(n)]


def _enc_body(x_ref, *refs):
    out_ref = refs[-1]
    p = _take_mlp(refs, 0)
    y = _ln_mlp(p, x_ref[...][:, :9])
    out_ref[...] = jnp.pad(y, ((0, 0), (0, 128 - H)))


def _encoder(x, params):
    xp = jnp.pad(x, ((0, 0), (0, 7)))
    blk = 2000
    return pl.pallas_call(
        _enc_body,
        grid=(E0 // blk,),
        in_specs=[pl.BlockSpec((blk, 16), lambda i: (i, 0))] + _wspecs(10),
        out_specs=pl.BlockSpec((blk, 128), lambda i: (i, 0)),
        out_shape=jax.ShapeDtypeStruct((E0, 128), jnp.float32),
    )(xp, *_mlp_args(params, "eb"))


def _mask_body(a_ref, b_ref, aij_ref, pk_ref, c16_ref, acc_ref):
    i, j, k = pl.program_id(0), pl.program_id(1), pl.program_id(2)

    @pl.when(k == 0)
    def _():
        acc_ref[...] = jnp.zeros_like(acc_ref)

    acc_ref[...] += jnp.dot(a_ref[...], b_ref[...],
                            preferred_element_type=jnp.float32)

    @pl.when(k == pl.num_programs(2) - 1)
    def _():
        blk = acc_ref.shape[0]
        rows = i * blk + lax.broadcasted_iota(jnp.int32, acc_ref.shape, 0)
        cols = j * blk + lax.broadcasted_iota(jnp.int32, acc_ref.shape, 1)
        two = (acc_ref[...] > 0.0).astype(jnp.int32) * (rows != cols).astype(jnp.int32)
        mi = jnp.minimum((aij_ref[...] > 0).astype(jnp.int32) + two, 1)
        lanemod = lax.broadcasted_iota(jnp.int32, acc_ref.shape, 1) % 16
        x = mi
        for sh in (1, 2, 4, 8):
            r = jnp.pad(x, ((0, 0), (sh, 0)))[:, :-sh]
            x = x + r * (lanemod >= sh).astype(jnp.int32)
        p_excl = x - mi
        c16_ref[...] = x.reshape(blk, blk // 16, 16)[:, :, 15][None]
        d = lanemod - p_excl
        packed = jnp.zeros_like(mi)
        for dd in range(16):
            src = cols * mi * (d == dd).astype(jnp.int32)
            if dd:
                src = jnp.pad(src, ((0, 0), (0, dd)))[:, dd:]
            packed = packed + src
        pk_ref[...] = packed


def _two_hop_mask(a_bf):
    blk = 512
    g = NP // blk
    return pl.pallas_call(
        _mask_body,
        grid=(g, g, g),
        in_specs=[
            pl.BlockSpec((blk, blk), lambda i, j, k: (i, k)),
            pl.BlockSpec((blk, blk), lambda i, j, k: (k, j)),
            pl.BlockSpec((blk, blk), lambda i, j, k: (i, j)),
        ],
        out_specs=[pl.BlockSpec((blk, blk), lambda i, j, k: (i, j)),
                   pl.BlockSpec((1, blk, blk // 16), lambda i, j, k: (j, i, 0))],
        out_shape=[jax.ShapeDtypeStruct((NP, NP), jnp.int32),
                   jax.ShapeDtypeStruct((NP // blk, NP, blk // 16), jnp.int32)],
        scratch_shapes=[pltpu.VMEM((blk, blk), jnp.float32)],
        compiler_params=pltpu.CompilerParams(
            dimension_semantics=("parallel", "arbitrary", "arbitrary")),
    )(a_bf, a_bf, a_bf)


def _compact_kernel(pk_hbm, c16_hbm, edges_hbm, cnts_hbm,
                    stage_v, row_v, cnt_v, t16_v):
    w = _wid()
    dump = jnp.full((16,), DUMPVAL, jnp.int32)
    nstage = HALF // 16

    def _init_stage():
        def bi(q, _):
            stage_v[pl.ds(q * 16, 16)] = dump
            return 0
        lax.fori_loop(0, nstage + 1, bi, 0)

    _init_stage()
    pltpu.sync_copy(stage_v.at[pl.ds(0, HALF)], edges_hbm.at[pl.ds(w * CAPW, HALF)])
    pltpu.sync_copy(stage_v.at[pl.ds(0, HALF)],
                    edges_hbm.at[pl.ds(w * CAPW + HALF, HALF)])

    def row_body(r, carry):
        gr = w * ROWS_W + r
        pltpu.sync_copy(pk_hbm.at[gr], row_v)
        pltpu.sync_copy(c16_hbm.at[gr], cnt_v)
        rowterm = gr << 14

        def c40_body(c40, carry2):
            cur2, nf2 = carry2
            cntv = cnt_v[pl.ds(c40 * 16, 16)]
            for u in range(16):
                cnt = cntv[u]
                off = (c40 * 16 + u) * 16
                v = row_v[pl.ds(off, 16)] + rowterm
                stage_v[pl.ds(cur2, 16)] = v
                cur3 = cur2 + cnt
                crossed = cur3 >= HALF

                def _flush():
                    carry_v = stage_v[pl.ds(HALF, 16)]

                    @pl.when(nf2 == 0)
                    def _():
                        pltpu.sync_copy(stage_v.at[pl.ds(0, HALF)],
                                        edges_hbm.at[pl.ds(w * CAPW, HALF)])

                    @pl.when(nf2 == 1)
                    def _():
                        pltpu.sync_copy(stage_v.at[pl.ds(0, HALF)],
                                        edges_hbm.at[pl.ds(w * CAPW + HALF, HALF)])

                    _init_stage()
                    stage_v[pl.ds(0, 16)] = carry_v

                pl.when(crossed)(_flush)
                cur2 = jnp.where(crossed, cur3 - HALF, cur3)
                nf2 = nf2 + crossed.astype(jnp.int32)
            return (cur2, nf2)

        return lax.fori_loop(0, NC16 // 16, c40_body, carry)

    cur, nf = lax.fori_loop(0, ROWS_W, row_body, (0, 0))
    stage_v[pl.ds(cur, 16)] = dump

    @pl.when(nf == 0)
    def _():
        pltpu.sync_copy(stage_v.at[pl.ds(0, HALF)],
                        edges_hbm.at[pl.ds(w * CAPW, HALF)])

    @pl.when(nf == 1)
    def _():
        pltpu.sync_copy(stage_v.at[pl.ds(0, HALF)],
                        edges_hbm.at[pl.ds(w * CAPW + HALF, HALF)])

    total = jnp.minimum(jnp.minimum(nf, 2) * HALF + cur, CAPW)
    t16_v[pl.ds(0, 16)] = jnp.zeros((16,), jnp.int32) + total
    pltpu.sync_copy(t16_v, cnts_hbm.at[w])


def _compact(pk, c16):
    k = functools.partial(
        pl.kernel,
        out_type=(jax.ShapeDtypeStruct((E_MAX + 8192,), jnp.int32),
                  jax.ShapeDtypeStruct((NW, 16), jnp.int32)),
        mesh=_mesh(),
        scratch_types=[pltpu.VMEM((HALF + 16,), jnp.int32),
                       pltpu.VMEM((NP,), jnp.int32),
                       pltpu.VMEM((NC16,), jnp.int32),
                       pltpu.VMEM((16,), jnp.int32)],
    )(_compact_kernel)
    return k(pk, c16)


def _extract_kernel(edges_hbm, g_hbm, e0_hbm, e1_hbm,
                    g_v, win0_v, win1_v, b0_v, b1_v):
    u = _wid()
    o0 = u * 5120
    pltpu.sync_copy(g_hbm, g_v)
    gs = []
    for q in range(3):
        gq = g_v[pl.ds(q * 16, 16)]
        for t in range(16):
            if q * 16 + t <= NW:
                gs.append(gq[t])
    w0 = jnp.int32(0)
    for jj in range(1, NW + 1):
        w0 = w0 + (gs[jj] <= o0).astype(jnp.int32)
    w0 = jnp.clip(w0, 0, NW - 1)
    w1 = jnp.minimum(w0 + 1, NW - 1)
    gw0 = jnp.int32(0)
    tstar = jnp.int32(0)
    for jj in range(NW + 1):
        gw0 = jnp.where(w0 == jj, gs[jj], gw0)
        tstar = jnp.where(w0 + 1 == jj, gs[jj], tstar)
    kstar = jnp.clip(tstar - o0, 0, 1 << 29)

    src0 = jnp.clip(w0 * CAPW + (o0 - gw0), 0, E_MAX + 8192 - 5136)
    src0a = pl.multiple_of((src0 >> 3) << 3, 8)
    shift0 = src0 - src0a
    pltpu.sync_copy(edges_hbm.at[pl.ds(src0a, 5136)], win0_v)
    src1 = pl.multiple_of(w1 * CAPW, 8)
    pltpu.sync_copy(edges_hbm.at[pl.ds(src1, 5136)], win1_v)

    def body(kk, _):
        off0 = shift0 + kk * 16
        off1 = jnp.clip(kk * 16 - kstar, 0, 5120)
        v0 = win0_v[pl.ds(off0, 16)]
        v1 = win1_v[pl.ds(off1, 16)]
        sel = (_iota16() + kk * 16) >= kstar
        v = jnp.where(sel, v1, v0)
        b0_v[pl.ds(kk * 16, 16)] = v >> 14
        b1_v[pl.ds(kk * 16, 16)] = v & 16383
        return 0

    lax.fori_loop(0, 320, body, 0)
    dst = pl.multiple_of(o0, 8)
    pltpu.sync_copy(b0_v, e0_hbm.at[pl.ds(dst, 5120)])
    pltpu.sync_copy(b1_v, e1_hbm.at[pl.ds(dst, 5120)])


def _extract(edges, g48):
    k = functools.partial(
        pl.kernel,
        out_type=(jax.ShapeDtypeStruct((E0P,), jnp.int32),
                  jax.ShapeDtypeStruct((E0P,), jnp.int32)),
        mesh=_mesh(),
        scratch_types=[pltpu.VMEM((48,), jnp.int32),
                       pltpu.VMEM((5136,), jnp.int32),
                       pltpu.VMEM((5136,), jnp.int32),
                       pltpu.VMEM((5120,), jnp.int32),
                       pltpu.VMEM((5120,), jnp.int32)],
    )(_extract_kernel)
    return k(edges, g48)


def _zero_spmem(acc_sh, z_v, rows_per_copy):
    sid = lax.axis_index("s")

    @pl.when(sid == 0)
    def _():
        zz = jnp.zeros((16,), jnp.float32)
        for r in range(rows_per_copy):
            for q in range(128 // 16):
                z_v[r, pl.ds(q * 16, 16)] = zz

        def cp(b, _):
            pltpu.sync_copy(z_v, acc_sh.at[pl.ds(b * rows_per_copy, rows_per_copy)])
            return 0
        lax.fori_loop(0, NP // rows_per_copy, cp, 0)

    plsc.subcore_barrier()


def _enc_scatter_kernel(xe_hbm, e0_hbm, e1_hbm, out_hbm,
                        idx_v, val_v, z_v, acc_sh):
    cid = lax.axis_index("c")
    w = _wid()
    _zero_spmem(acc_sh, z_v, 64)
    base = w * (E0 // NW)

    def batch(b, _):
        pltpu.sync_copy(xe_hbm.at[pl.ds(base + b * 40, 40)], val_v)
        pltpu.sync_copy(e0_hbm.at[pl.ds(base + b * 40, 40)], idx_v)
        pltpu.sync_copy(val_v, acc_sh.at[idx_v], add=True)
        pltpu.sync_copy(e1_hbm.at[pl.ds(base + b * 40, 40)], idx_v)
        pltpu.sync_copy(val_v, acc_sh.at[idx_v], add=True)
        return 0

    lax.fori_loop(0, E0 // NW // 40, batch, 0)
    plsc.subcore_barrier()
    sid = lax.axis_index("s")

    @pl.when(sid == 0)
    def _():
        pltpu.sync_copy(acc_sh, out_hbm.at[cid])


def _enc_scatter(xe, e0, e1):
    k = functools.partial(
        pl.kernel,
        out_type=jax.ShapeDtypeStruct((2, NP, 128), jnp.float32),
        mesh=_mesh(),
        scratch_types=[pltpu.VMEM((40,), jnp.int32),
                       pltpu.VMEM((40, 128), jnp.float32),
                       pltpu.VMEM((64, 128), jnp.float32),
                       pltpu.VMEM_SHARED((NP, 128), jnp.float32)],
    )(_enc_scatter_kernel)
    return k(xe, e0, e1)


def _pair_gather_kernel(tab_hbm, edges_hbm, fa_hbm, fb_hbm,
                        pk_v, ia_v, ib_v, ga_v, gb_v, sem):
    w = _wid()
    B = 128
    nbat = CAPW // B
    niter = (nbat + 2) // 3

    def batch3(bb, _):
        descs = []
        bases = []
        for k in range(3):
            b = jnp.minimum(bb * 3 + k, nbat - 1)
            base = pl.multiple_of(w * CAPW + b * B, 8)
            bases.append(base)
            pltpu.sync_copy(edges_hbm.at[pl.ds(base, B)], pk_v.at[k])

            def unpk(q, _2, k=k):
                v = pk_v[k, pl.ds(q * 16, 16)]
                ia_v[k, pl.ds(q * 16, 16)] = v >> 14
                ib_v[k, pl.ds(q * 16, 16)] = v & 16383
                return 0
            lax.fori_loop(0, B // 16, unpk, 0)
            descs.append(pltpu.async_copy(tab_hbm.at[ia_v.at[k]], ga_v.at[k], sem))
            descs.append(pltpu.async_copy(tab_hbm.at[ib_v.at[k]], gb_v.at[k], sem))
        for d in descs:
            d.wait()
        for k in range(3):
            pltpu.sync_copy(ga_v.at[k], fa_hbm.at[pl.ds(bases[k], B)])
            pltpu.sync_copy(gb_v.at[k], fb_hbm.at[pl.ds(bases[k], B)])
        return 0

    lax.fori_loop(0, niter, batch3, 0)


def _pair_gather(tab, edges):
    k = functools.partial(
        pl.kernel,
        out_type=(jax.ShapeDtypeStruct((E_MAX, 128), jnp.float32),
                  jax.ShapeDtypeStruct((E_MAX, 128), jnp.float32)),
        mesh=_mesh(),
        scratch_types=[pltpu.VMEM((3, 128), jnp.int32),
                       pltpu.VMEM((3, 128), jnp.int32),
                       pltpu.VMEM((3, 128), jnp.int32),
                       pltpu.VMEM((3, 128, 128), jnp.float32),
                       pltpu.VMEM((3, 128, 128), jnp.float32),
                       pltpu.SemaphoreType.DMA],
    )(_pair_gather_kernel)
    return k(tab, edges)


def _pair_scatter_kernel(xew_hbm, edges_hbm, out_hbm,
                         pk_v, ia_v, ib_v, val_v, z_v, acc_sh, sem):
    cid = lax.axis_index("c")
    w = _wid()
    _zero_spmem(acc_sh, z_v, 64)
    B = 256

    def batch(b, _):
        base = w * CAPW + b * B
        pltpu.sync_copy(xew_hbm.at[pl.ds(base, B)], val_v)
        pltpu.sync_copy(edges_hbm.at[pl.ds(base, B)], pk_v)

        for ss in range(B // 64):
            for q4 in range(4):
                v = pk_v[pl.ds(ss * 64 + q4 * 16, 16)]
                ia_v[ss, pl.ds(q4 * 16, 16)] = v >> 14
                ib_v[ss, pl.ds(q4 * 16, 16)] = v & 16383

        descs = []
        for ss in range(B // 64):
            descs.append(pltpu.async_copy(val_v.at[pl.ds(ss * 64, 64)],
                                          acc_sh.at[ia_v.at[ss]], sem, add=True))
            descs.append(pltpu.async_copy(val_v.at[pl.ds(ss * 64, 64)],
                                          acc_sh.at[ib_v.at[ss]], sem, add=True))
        for d in descs:
            d.wait()
        return 0

    lax.fori_loop(0, CAPW // B, batch, 0)
    plsc.subcore_barrier()
    sid = lax.axis_index("s")

    @pl.when(sid == 0)
    def _():
        pltpu.sync_copy(acc_sh, out_hbm.at[cid])


def _pair_scatter(xew, edges):
    k = functools.partial(
        pl.kernel,
        out_type=jax.ShapeDtypeStruct((2, NP, 128), jnp.float32),
        mesh=_mesh(),
        scratch_types=[pltpu.VMEM((256,), jnp.int32),
                       pltpu.VMEM((4, 64), jnp.int32),
                       pltpu.VMEM((4, 64), jnp.int32),
                       pltpu.VMEM((256, 128), jnp.float32),
                       pltpu.VMEM((64, 128), jnp.float32),
                       pltpu.VMEM_SHARED((NP, 128), jnp.float32),
                       pltpu.SemaphoreType.DMA],
    )(_pair_scatter_kernel)
    return k(xew, edges)


def _dec_gather_kernel(tab_hbm, e0_hbm, e1_hbm, fa_hbm, fb_hbm,
                       ia_v, ib_v, ga_v, gb_v, sem):
    u = _wid()
    B = 128
    nbat = 5120 // B
    niter = (nbat + 2) // 3

    def batch3(bb, _):
        descs = []
        bases = []
        for k in range(3):
            b = jnp.minimum(bb * 3 + k, nbat - 1)
            base = pl.multiple_of(u * 5120 + b * B, 8)
            bases.append(base)
            pltpu.sync_copy(e0_hbm.at[pl.ds(base, B)], ia_v.at[k])
            pltpu.sync_copy(e1_hbm.at[pl.ds(base, B)], ib_v.at[k])
            descs.append(pltpu.async_copy(tab_hbm.at[ia_v.at[k]], ga_v.at[k], sem))
            descs.append(pltpu.async_copy(tab_hbm.at[ib_v.at[k]], gb_v.at[k], sem))
        for d in descs:
            d.wait()
        for k in range(3):
            pltpu.sync_copy(ga_v.at[k], fa_hbm.at[pl.ds(bases[k], B)])
            pltpu.sync_copy(gb_v.at[k], fb_hbm.at[pl.ds(bases[k], B)])
        return 0

    lax.fori_loop(0, niter, batch3, 0)


def _dec_gather(tab, e0, e1):
    k = functools.partial(
        pl.kernel,
        out_type=(jax.ShapeDtypeStruct((E0P, 128), jnp.float32),
                  jax.ShapeDtypeStruct((E0P, 128), jnp.float32)),
        mesh=_mesh(),
        scratch_types=[pltpu.VMEM((3, 128), jnp.int32),
                       pltpu.VMEM((3, 128), jnp.int32),
                       pltpu.VMEM((3, 128, 128), jnp.float32),
                       pltpu.VMEM((3, 128, 128), jnp.float32),
                       pltpu.SemaphoreType.DMA],
    )(_dec_gather_kernel)
    return k(tab, e0, e1)


def _node_body(acc_ref, xn_ref, *refs):
    out_ref = refs[-1]
    p = _take_mlp(refs, 0)
    agg = acc_ref[0] + acc_ref[1]
    xn = xn_ref[...][:, :H]
    y = _ln_mlp(p, jnp.concatenate([xn, agg[:, :H]], axis=1))
    out_ref[...] = jnp.pad(xn + y, ((0, 0), (0, 128 - H)))


def _node_mlp(acc, xn_tab, params, name):
    blk = 1024
    return pl.pallas_call(
        _node_body,
        grid=(NP // blk,),
        in_specs=[pl.BlockSpec((2, blk, 128), lambda i: (0, i, 0)),
                  pl.BlockSpec((blk, 128), lambda i: (i, 0))] + _wspecs(10),
        out_specs=pl.BlockSpec((blk, 128), lambda i: (i, 0)),
        out_shape=jax.ShapeDtypeStruct((NP, 128), jnp.float32),
    )(acc, xn_tab, *_mlp_args(params, name))


def _node0_body(acc_ref, *refs):
    out_ref = refs[-1]
    p = _take_mlp(refs, 0)
    pre = acc_ref[0][:, :H] + acc_ref[1][:, :H]
    y = _ln_mlp(p, pre)
    out_ref[...] = jnp.pad(y, ((0, 0), (0, 128 - H)))


def _node0_mlp(acc, params):
    blk = 1024
    return pl.pallas_call(
        _node0_body,
        grid=(NP // blk,),
        in_specs=[pl.BlockSpec((2, blk, 128), lambda i: (0, i, 0))] + _wspecs(10),
        out_specs=pl.BlockSpec((blk, 128), lambda i: (i, 0)),
        out_shape=jax.ShapeDtypeStruct((NP, 128), jnp.float32),
    )(acc, *_mlp_args(params, "nb"))


def _pair0_body(fa_ref, fb_ref, *refs):
    xew_ref, xe1_ref = refs[-2], refs[-1]
    p1 = _take_mlp(refs, 0)
    p2 = _take_mlp(refs, 10)
    a = fa_ref[...][:, :H]
    b = fb_ref[...][:, :H]
    ab = jnp.concatenate([a, b], axis=1)
    xe0 = _ln_mlp(p1, ab)
    xe_new = _ln_mlp(p2, jnp.concatenate([ab, xe0], axis=1))
    xew_ref[...] = jnp.pad(xe_new, ((0, 0), (0, 128 - H)))
    xe1_ref[...] = xe0 + xe_new


def _pair0(fa, fb, params):
    blk = 2048
    return pl.pallas_call(
        _pair0_body,
        grid=(E_MAX // blk,),
        in_specs=[pl.BlockSpec((blk, 128), lambda i: (i, 0)),
                  pl.BlockSpec((blk, 128), lambda i: (i, 0))] + _wspecs(20),
        out_specs=[pl.BlockSpec((blk, 128), lambda i: (i, 0)),
                   pl.BlockSpec((blk, H), lambda i: (i, 0))],
        out_shape=[jax.ShapeDtypeStruct((E_MAX, 128), jnp.float32),
                   jax.ShapeDtypeStruct((E_MAX, H), jnp.float32)],
    )(fa, fb, *_mlp_args(params, "eb1"), *_mlp_args(params, "gn_eb_0"))


def _pair1_body(fa_ref, fb_ref, xe1_ref, *refs):
    xew_ref = refs[-1]
    p = _take_mlp(refs, 0)
    a = fa_ref[...][:, :H]
    b = fb_ref[...][:, :H]
    xe_new = _ln_mlp(p, jnp.concatenate([a, b, xe1_ref[...]], axis=1))
    xew_ref[...] = jnp.pad(xe_new, ((0, 0), (0, 128 - H)))


def _pair1(fa, fb, xe1, params):
    blk = 2048
    return pl.pallas_call(
        _pair1_body,
        grid=(E_MAX // blk,),
        in_specs=[pl.BlockSpec((blk, 128), lambda i: (i, 0)),
                  pl.BlockSpec((blk, 128), lambda i: (i, 0)),
                  pl.BlockSpec((blk, H), lambda i: (i, 0))] + _wspecs(10),
        out_specs=pl.BlockSpec((blk, 128), lambda i: (i, 0)),
        out_shape=jax.ShapeDtypeStruct((E_MAX, 128), jnp.float32),
    )(fa, fb, xe1, *_mlp_args(params, "gn_eb_1"))


def _dec_body(fa_ref, fb_ref, *refs):
    out_ref = refs[-1]
    p = dict(_take_mlp(refs, 0))
    a = fa_ref[...][:, :H]
    b = fb_ref[...][:, :H]
    x = jnp.concatenate([a, b], axis=1)
    x = jnp.maximum(jnp.dot(x, p["W1"], preferred_element_type=jnp.float32) + p["b1"], 0.0)
    x = jnp.maximum(jnp.dot(x, p["W2"], preferred_element_type=jnp.float32) + p["b2"], 0.0)
    x = jnp.maximum(jnp.dot(x, p["W3"], preferred_element_type=jnp.float32) + p["b3"], 0.0)
    x = jnp.dot(x, p["W4"], preferred_element_type=jnp.float32) + p["b4"]
    y3 = x[:, :3]
    mu = jnp.mean(y3, axis=-1, keepdims=True)
    var = jnp.mean((y3 - mu) ** 2, axis=-1, keepdims=True)
    y = (y3 - mu) * jax.lax.rsqrt(var + 1e-5) * p["g"][:, :3] + p["be"][:, :3]
    out_ref[...] = jnp.pad(y, ((0, 0), (0, 5)))


def _decoder(fa, fb, params):
    p = params["dec"]
    args = [p["W1"], p["b1"].reshape(1, -1), p["W2"], p["b2"].reshape(1, -1),
            p["W3"], p["b3"].reshape(1, -1),
            jnp.pad(p["W4"], ((0, 0), (0, 5))), jnp.pad(p["b4"], (0, 5)).reshape(1, -1),
            jnp.pad(p["g"], (0, 5)).reshape(1, -1), jnp.pad(p["be"], (0, 5)).reshape(1, -1)]
    blk = 2048
    return pl.pallas_call(
        _dec_body,
        grid=(E0P // blk,),
        in_specs=[pl.BlockSpec((blk, 128), lambda i: (i, 0)),
                  pl.BlockSpec((blk, 128), lambda i: (i, 0))] + _wspecs(10),
        out_specs=pl.BlockSpec((blk, 8), lambda i: (i, 0)),
        out_shape=jax.ShapeDtypeStruct((E0P, 8), jnp.float32),
    )(fa, fb, *args)


def kernel(x, edge_index, num_nodes, params):
    src = edge_index[0].astype(jnp.int32)
    dst = edge_index[1].astype(jnp.int32)

    a_bf = jnp.zeros((NP, NP), jnp.bfloat16).at[src, dst].set(jnp.bfloat16(1))

    pk, c16 = _two_hop_mask(a_bf)
    c16f = jnp.moveaxis(c16, 0, 1).reshape(NP, NC16)
    edges, cnts16 = _compact(pk, c16f)
    cnts = cnts16[:, 0]
    g = jnp.concatenate([jnp.zeros((1,), jnp.int32), jnp.cumsum(cnts)])
    nnz = g[NW]
    g48 = jnp.concatenate([g.astype(jnp.int32),
                           jnp.full((48 - NW - 1,), 1 << 30, jnp.int32)])
    e0p, e1p = _extract(edges, g48)
    guard = jnp.arange(E0P, dtype=jnp.int32) < nnz
    e0p = jnp.where(guard, e0p, 0)
    e1p = jnp.where(guard, e1p, 0)

    xe = _encoder(x, params)
    xn_pre = _enc_scatter(xe, e0p[:E0], e1p[:E0])
    xn_tab = _node0_mlp(xn_pre, params)

    fa0, fb0 = _pair_gather(xn_tab, edges)
    xew0, xe1c = _pair0(fa0, fb0, params)
    acc0 = _pair_scatter(xew0, edges)
    xn1_tab = _node_mlp(acc0, xn_tab, params, "gn_nb_0")

    fa1, fb1 = _pair_gather(xn1_tab, edges)
    xew1 = _pair1(fa1, fb1, xe1c, params)
    acc1 = _pair_scatter(xew1, edges)
    xn2_tab = _node_mlp(acc1, xn1_tab, params, "gn_nb_1")

    dfa, dfb = _dec_gather(xn2_tab, e0p, e1p)
    outp = _decoder(dfa, dfb, params)
    return outp[:E0, :3]
